# baseline (speedup 1.0000x reference)
import jax
import jax.numpy as jnp
from jax.experimental import pallas as pl

N = 4096
D = 256
K = 14
BR = 1024
NUM_BLOCKS = N // BR
SPREAD = 1.0


def _bf16_round(v):
    u = jax.lax.bitcast_convert_type(v, jnp.uint32)
    r = (u + jnp.uint32(0x7FFF) + ((u >> 16) & jnp.uint32(1))) \
        & jnp.uint32(0xFFFF0000)
    return jax.lax.bitcast_convert_type(r, jnp.float32)


def _loss_block(x_blk_ref, x_all_ref, x2_blk_ref, x2_all_ref, e2_ref,
                e2t_ref, eb0_ref, eb1_ref, eb0t_ref, eb1t_ref, out_ref):
    i = pl.program_id(0)

    x = x_blk_ref[...]
    xa = x_all_ref[...]
    x2_blk = x2_blk_ref[...]
    x2_all = x2_all_ref[...]
    xy = jax.lax.dot_general(x, xa, (((1,), (1,)), ((), ())),
                             preferred_element_type=jnp.float32)
    d2 = jnp.maximum(x2_blk + x2_all - 2.0 * xy, 1e-12)

    iota = jax.lax.broadcasted_iota(jnp.int32, (BR, N), 1)
    key = jax.lax.bitcast_convert_type(
        (jax.lax.bitcast_convert_type(d2, jnp.int32) & jnp.int32(~0xFFF))
        | iota, jnp.float32)
    inf = jnp.float32(jnp.inf)
    for _ in range(K):
        mkey = jnp.min(key, axis=1, keepdims=True)
        key = jnp.where(key == mkey, inf, key)
    selected = key == inf

    d = jnp.sqrt(d2)
    hs = jnp.exp(-d / SPREAD)

    ld2 = (e2_ref[...] + e2t_ref[...]) \
        - 2.0 * (eb0_ref[...] * eb0t_ref[...] + eb1_ref[...] * eb1t_ref[...])
    ld = jnp.sqrt(jnp.maximum(ld2, 1e-12))
    ls = 1.0 / (1.0 + ld)

    contrib = jnp.where(selected,
                        hs * jnp.log(ls + 1e-10),
                        (1.0 - hs) * jnp.log(1.0 - ls + 1e-10))
    total = jnp.sum(jnp.sum(contrib, axis=1, keepdims=True))

    @pl.when(i == 0)
    def _():
        out_ref[...] = jnp.zeros((1, 1), jnp.float32)

    out_ref[...] += total.reshape(1, 1)

    @pl.when(i == NUM_BLOCKS - 1)
    def _():
        out_ref[...] = out_ref[...] * (-100.0 / (N * N))


@jax.jit
def kernel(high_dim_data, low_dim_embedding):
    x = high_dim_data.astype(jnp.float32)
    e = low_dim_embedding.astype(jnp.float32)
    x2_col = jnp.sum(x * x, axis=1, keepdims=True)
    x2_all = x2_col.reshape(1, N)
    xb = x.astype(jnp.bfloat16)
    e2 = jnp.sum(e * e, axis=1, keepdims=True)
    e2t = e2.reshape(1, N)
    eb = _bf16_round(e)
    eb0 = eb[:, 0:1]
    eb1 = eb[:, 1:2]
    eb0t = eb0.reshape(1, N)
    eb1t = eb1.reshape(1, N)

    out = pl.pallas_call(
        _loss_block,
        grid=(NUM_BLOCKS,),
        in_specs=[
            pl.BlockSpec((BR, D), lambda i: (i, 0)),
            pl.BlockSpec((N, D), lambda i: (0, 0)),
            pl.BlockSpec((BR, 1), lambda i: (i, 0)),
            pl.BlockSpec((1, N), lambda i: (0, 0)),
            pl.BlockSpec((BR, 1), lambda i: (i, 0)),
            pl.BlockSpec((1, N), lambda i: (0, 0)),
            pl.BlockSpec((BR, 1), lambda i: (i, 0)),
            pl.BlockSpec((BR, 1), lambda i: (i, 0)),
            pl.BlockSpec((1, N), lambda i: (0, 0)),
            pl.BlockSpec((1, N), lambda i: (0, 0)),
        ],
        out_specs=pl.BlockSpec((1, 1), lambda i: (0, 0)),
        out_shape=jax.ShapeDtypeStruct((1, 1), jnp.float32),
    )(xb, xb, x2_col, x2_all, e2, e2t, eb0, eb1, eb0t, eb1t)
    return out[0, 0]


# speedup vs baseline: 1.2530x; 1.2530x over previous
import jax
import jax.numpy as jnp
from jax.experimental import pallas as pl

N = 4096
D = 256
K = 14
BR = 512
NUM_BLOCKS = N // BR
SPREAD = 1.0


def _bf16_round(v):
    u = jax.lax.bitcast_convert_type(v, jnp.uint32)
    r = (u + jnp.uint32(0x7FFF) + ((u >> 16) & jnp.uint32(1))) \
        & jnp.uint32(0xFFFF0000)
    return jax.lax.bitcast_convert_type(r, jnp.float32)


def _loss_block(x_blk_ref, x_all_ref, x2_blk_ref, x2_all_ref, e2_ref,
                e2t_ref, eb0_ref, eb1_ref, eb0t_ref, eb1t_ref, out_ref):
    i = pl.program_id(0)

    x = x_blk_ref[...]
    xa = x_all_ref[...]
    x2_blk = x2_blk_ref[...]
    x2_all = x2_all_ref[...]
    xy = jax.lax.dot_general(x, xa, (((1,), (1,)), ((), ())),
                             preferred_element_type=jnp.float32)
    d2 = jnp.maximum(x2_blk + x2_all - 2.0 * xy, 1e-12)

    iota = jax.lax.broadcasted_iota(jnp.int32, (BR, N), 1)
    key = jax.lax.bitcast_convert_type(
        (jax.lax.bitcast_convert_type(d2, jnp.int32) & jnp.int32(~0xFFF))
        | iota, jnp.float32)
    inf = jnp.float32(jnp.inf)
    for _ in range(K):
        mkey = jnp.min(key, axis=1, keepdims=True)
        key = jnp.where(key == mkey, inf, key)
    selected = key == inf

    d = jnp.sqrt(d2)
    hs = jnp.exp(-d / SPREAD)

    ld2 = (e2_ref[...] + e2t_ref[...]) \
        - 2.0 * (eb0_ref[...] * eb0t_ref[...] + eb1_ref[...] * eb1t_ref[...])
    ld = jnp.sqrt(jnp.maximum(ld2, 1e-12))
    ls = 1.0 / (1.0 + ld)

    contrib = jnp.where(selected,
                        hs * jnp.log(ls + 1e-10),
                        (1.0 - hs) * jnp.log(1.0 - ls + 1e-10))
    total = jnp.sum(jnp.sum(contrib, axis=1, keepdims=True))

    @pl.when(i == 0)
    def _():
        out_ref[...] = jnp.zeros((1, 1), jnp.float32)

    out_ref[...] += total.reshape(1, 1)

    @pl.when(i == NUM_BLOCKS - 1)
    def _():
        out_ref[...] = out_ref[...] * (-100.0 / (N * N))


@jax.jit
def kernel(high_dim_data, low_dim_embedding):
    x = high_dim_data.astype(jnp.float32)
    e = low_dim_embedding.astype(jnp.float32)
    x2_col = jnp.sum(x * x, axis=1, keepdims=True)
    x2_all = x2_col.reshape(1, N)
    xb = x.astype(jnp.bfloat16)
    e2 = jnp.sum(e * e, axis=1, keepdims=True)
    e2t = e2.reshape(1, N)
    eb = _bf16_round(e)
    eb0 = eb[:, 0:1]
    eb1 = eb[:, 1:2]
    eb0t = eb0.reshape(1, N)
    eb1t = eb1.reshape(1, N)

    out = pl.pallas_call(
        _loss_block,
        grid=(NUM_BLOCKS,),
        in_specs=[
            pl.BlockSpec((BR, D), lambda i: (i, 0)),
            pl.BlockSpec((N, D), lambda i: (0, 0)),
            pl.BlockSpec((BR, 1), lambda i: (i, 0)),
            pl.BlockSpec((1, N), lambda i: (0, 0)),
            pl.BlockSpec((BR, 1), lambda i: (i, 0)),
            pl.BlockSpec((1, N), lambda i: (0, 0)),
            pl.BlockSpec((BR, 1), lambda i: (i, 0)),
            pl.BlockSpec((BR, 1), lambda i: (i, 0)),
            pl.BlockSpec((1, N), lambda i: (0, 0)),
            pl.BlockSpec((1, N), lambda i: (0, 0)),
        ],
        out_specs=pl.BlockSpec((1, 1), lambda i: (0, 0)),
        out_shape=jax.ShapeDtypeStruct((1, 1), jnp.float32),
    )(xb, xb, x2_col, x2_all, e2, e2t, eb0, eb1, eb0t, eb1t)
    return out[0, 0]
